# baseline (speedup 1.0000x reference)
import functools

import jax
import jax.numpy as jnp
from jax import lax
from jax.experimental import pallas as pl
from jax.experimental.pallas import tpu as pltpu
from jax.experimental.pallas import tpu_sc as plsc

N = 10000
E = 320000
D = 128
NC = 2
NS = 16
CHUNK = 64
NR = 5
NI = 2 * NR
LAG = NR - 1
TOTAL_CHUNKS = E // CHUNK
CHUNKS_PER_TILE = 320
HALF = 5120
ROWS_PER_TILE = HALF // NS
LAST_ROWS = N - (NC * HALF - ROWS_PER_TILE)


def _mlp_body(x_ref, w1a_ref, w1b_ref, s_ref, b1_ref, w2_ref, b2_ref,
              o_ref, z_ref):
    b1eff = b1_ref[...] + jnp.dot(s_ref[...], w1b_ref[...],
                                  preferred_element_type=jnp.float32)
    h = jnp.dot(x_ref[...], w1a_ref[...], preferred_element_type=jnp.float32)
    h = jnp.maximum(h + b1eff, 0.0)
    o = jnp.dot(h, w2_ref[...], preferred_element_type=jnp.float32)
    o_ref[...] = o + b2_ref[...]
    z_ref[...] = jnp.zeros_like(z_ref)


def _node_mlp(x, scalars, W1, b1, W2, b2):
    blk = 10000
    grid = N // blk
    zblk = ROWS_PER_TILE // grid
    full = lambda shape: pl.BlockSpec(shape, lambda i: (0,) * len(shape))
    return pl.pallas_call(
        _mlp_body,
        grid=(grid,),
        in_specs=[
            pl.BlockSpec((blk, D), lambda i: (i, 0)),
            full((D, D)),
            full((4, D)),
            full((1, 4)),
            full((1, D)),
            full((D, D)),
            full((1, D)),
        ],
        out_specs=[pl.BlockSpec((blk, D), lambda i: (i, 0)),
                   pl.BlockSpec((zblk, D), lambda i: (i, 0))],
        out_shape=[jax.ShapeDtypeStruct((N, D), jnp.float32),
                   jax.ShapeDtypeStruct((ROWS_PER_TILE, D), jnp.float32)],
    )(x, W1[:D], W1[D:], scalars, b1.reshape(1, D), W2, b2.reshape(1, D))


def _gid(s, chunk):
    return s + NS * chunk


def _idx_copies(eidx_hbm, s, chunk, idx_v, q, sem_s, sem_d):
    off = jnp.minimum(_gid(s, chunk), TOTAL_CHUNKS - 1) * CHUNK
    return (
        pltpu.make_async_copy(eidx_hbm.at[0, pl.ds(off, CHUNK)],
                              idx_v.at[q, 0], sem_s),
        pltpu.make_async_copy(eidx_hbm.at[1, pl.ds(off, CHUNK)],
                              idx_v.at[q, 1], sem_d),
    )


def _remap(idx_v, q, lo, lim):
    for j in range(CHUNK // 16):
        sl = pl.ds(j * 16, 16)
        srcv = idx_v[q, 0, sl]
        dl = idx_v[q, 1, sl] - lo
        ok = (dl >= 0) & (dl < lim)
        neg1 = jnp.full((16,), -1, jnp.int32)
        idx_v[q, 0, sl] = jnp.where(ok, srcv, neg1)
        idx_v[q, 1, sl] = jnp.where(ok, dl, neg1)


def _gather_idx(idx_v, q):
    return plsc.Indices(idx_v.at[q, 0], ignored_value=-1)


def _scatter_idx(idx_v, q):
    return plsc.Indices(idx_v.at[q, 1], ignored_value=-1)


def _sc_body(msg_hbm, eidx_hbm, zeros_hbm, out_hbm, idx_v, bufs, acc,
             semi, semi2, semr, semw):
    c = lax.axis_index("c")
    s = lax.axis_index("s")
    lo = c * HALF

    for q in range(NR):
        for d in _idx_copies(eidx_hbm, s, q, idx_v, q, semi[q], semi2[q]):
            d.start()

    base = s * ROWS_PER_TILE
    pltpu.sync_copy(zeros_hbm, acc.at[pl.ds(base, ROWS_PER_TILE)])
    plsc.subcore_barrier()

    @pl.loop(0, CHUNKS_PER_TILE, step=NI)
    def _edges(g):
        for i in range(NI):
            chunk = g + i
            b = i % NR

            @pl.when(chunk >= NR)
            def _():
                pltpu.make_async_copy(
                    bufs[b], acc.at[_scatter_idx(idx_v, (i + NR) % NI)],
                    semw[b]).wait()

            for d in _idx_copies(eidx_hbm, s, chunk, idx_v, i,
                                 semi[i], semi2[i]):
                d.wait()
            lim = jnp.where(_gid(s, chunk) < TOTAL_CHUNKS, HALF, 0)
            _remap(idx_v, i, lo, lim)
            pltpu.async_copy(msg_hbm.at[_gather_idx(idx_v, i)], bufs[b],
                             semr[b])

            @pl.when(chunk >= LAG)
            def _():
                bs = (i + 1) % NR
                qs = (i + NI - LAG) % NI
                pltpu.make_async_copy(msg_hbm.at[_gather_idx(idx_v, qs)],
                                      bufs[bs], semr[bs]).wait()
                pltpu.async_copy(bufs[bs], acc.at[_scatter_idx(idx_v, qs)],
                                 semw[bs], add=True)

            @pl.when(chunk + NR < CHUNKS_PER_TILE)
            def _():
                q2 = (i + NR) % NI
                for d in _idx_copies(eidx_hbm, s, chunk + NR, idx_v, q2,
                                     semi[q2], semi2[q2]):
                    d.start()

    for t in range(LAG):
        ct = CHUNKS_PER_TILE - LAG + t
        bs = ct % NR
        qs = ct % NI
        pltpu.make_async_copy(msg_hbm.at[_gather_idx(idx_v, qs)],
                              bufs[bs], semr[bs]).wait()
        pltpu.sync_copy(bufs[bs], acc.at[_scatter_idx(idx_v, qs)], add=True)
    cl = CHUNKS_PER_TILE - LAG - 1
    pltpu.make_async_copy(bufs[cl % NR],
                          acc.at[_scatter_idx(idx_v, cl % NI)],
                          semw[cl % NR]).wait()
    plsc.subcore_barrier()

    gbase = lo + base

    @pl.when(gbase + ROWS_PER_TILE <= N)
    def _():
        pltpu.sync_copy(acc.at[pl.ds(base, ROWS_PER_TILE)],
                        out_hbm.at[pl.ds(gbase, ROWS_PER_TILE)])

    @pl.when(gbase + ROWS_PER_TILE > N)
    def _():
        pltpu.sync_copy(acc.at[pl.ds(base, LAST_ROWS)],
                        out_hbm.at[pl.ds(gbase, LAST_ROWS)])


def _sc_aggregate(node_msg, eidx, zeros):
    mesh = plsc.VectorSubcoreMesh(core_axis_name="c", subcore_axis_name="s",
                                  num_cores=NC)
    k = pl.kernel(
        _sc_body,
        mesh=mesh,
        out_type=jax.ShapeDtypeStruct((N, D), jnp.float32),
        scratch_types=[
            pltpu.VMEM((NI, 2, CHUNK), jnp.int32),
            [pltpu.VMEM((CHUNK, D), jnp.float32)] * NR,
            pltpu.VMEM_SHARED((HALF, D), jnp.float32),
            [pltpu.SemaphoreType.DMA] * NI,
            [pltpu.SemaphoreType.DMA] * NI,
            [pltpu.SemaphoreType.DMA] * NR,
            [pltpu.SemaphoreType.DMA] * NR,
        ],
    )
    return k(node_msg, eidx, zeros)


def kernel(x, scalars, edge_index, W1, b1, W2, b2):
    node_msg, zeros = _node_mlp(x, scalars, W1, b1, W2, b2)
    eidx = edge_index.astype(jnp.int32)
    return _sc_aggregate(node_msg, eidx, zeros)


# speedup vs baseline: 1.2355x; 1.2355x over previous
import functools

import jax
import jax.numpy as jnp
from jax import lax
from jax.experimental import pallas as pl
from jax.experimental.pallas import tpu as pltpu
from jax.experimental.pallas import tpu_sc as plsc

N = 10000
E = 320000
D = 128
NC = 2
NS = 16
NW = NC * NS
CHUNK = 64
NR = 5
NI = 2 * NR
LAG = NR - 1
TOTAL_CHUNKS = E // CHUNK
CHUNKS_PER_TILE = 160
ACC_ROWS = 10112
ROWS_PER_TILE = ACC_ROWS // NS
LAST_ROWS = N - (NS - 1) * ROWS_PER_TILE


def _mlp_body(x_ref, w1a_ref, w1b_ref, s_ref, b1_ref, w2_ref, b2_ref,
              o_ref, z_ref):
    b1eff = b1_ref[...] + jnp.dot(s_ref[...], w1b_ref[...],
                                  preferred_element_type=jnp.float32)
    h = jnp.dot(x_ref[...], w1a_ref[...], preferred_element_type=jnp.float32)
    h = jnp.maximum(h + b1eff, 0.0)
    o = jnp.dot(h, w2_ref[...], preferred_element_type=jnp.float32)
    o_ref[...] = o + b2_ref[...]
    z_ref[...] = jnp.zeros_like(z_ref)


def _node_mlp(x, scalars, W1, b1, W2, b2):
    blk = 10000
    grid = N // blk
    zblk = ROWS_PER_TILE // grid
    full = lambda shape: pl.BlockSpec(shape, lambda i: (0,) * len(shape))
    return pl.pallas_call(
        _mlp_body,
        grid=(grid,),
        in_specs=[
            pl.BlockSpec((blk, D), lambda i: (i, 0)),
            full((D, D)),
            full((4, D)),
            full((1, 4)),
            full((1, D)),
            full((D, D)),
            full((1, D)),
        ],
        out_specs=[pl.BlockSpec((blk, D), lambda i: (i, 0)),
                   pl.BlockSpec((zblk, D), lambda i: (i, 0))],
        out_shape=[jax.ShapeDtypeStruct((N, D), jnp.float32),
                   jax.ShapeDtypeStruct((ROWS_PER_TILE, D), jnp.float32)],
    )(x, W1[:D], W1[D:], scalars, b1.reshape(1, D), W2, b2.reshape(1, D))


def _gid(wid, chunk):
    return wid + NW * chunk


def _idx_copies(eidx_hbm, wid, chunk, idx_v, q, sem_s, sem_d):
    off = jnp.minimum(_gid(wid, chunk), TOTAL_CHUNKS - 1) * CHUNK
    return (
        pltpu.make_async_copy(eidx_hbm.at[0, pl.ds(off, CHUNK)],
                              idx_v.at[q, 0], sem_s),
        pltpu.make_async_copy(eidx_hbm.at[1, pl.ds(off, CHUNK)],
                              idx_v.at[q, 1], sem_d),
    )


def _mask_invalid(idx_v, q):
    neg1 = jnp.full((16,), -1, jnp.int32)
    for d in range(2):
        for j in range(CHUNK // 16):
            idx_v[q, d, pl.ds(j * 16, 16)] = neg1


def _gather_idx(idx_v, q):
    return plsc.Indices(idx_v.at[q, 0], ignored_value=-1)


def _scatter_idx(idx_v, q):
    return plsc.Indices(idx_v.at[q, 1], ignored_value=-1)


def _sc_body(msg_hbm, eidx_hbm, zeros_hbm, outp_hbm, idx_v, bufs, acc,
             semi, semi2, semr, semw):
    c = lax.axis_index("c")
    s = lax.axis_index("s")
    wid = s * NC + c

    for q in range(NR):
        for d in _idx_copies(eidx_hbm, wid, q, idx_v, q, semi[q], semi2[q]):
            d.start()

    base = s * ROWS_PER_TILE
    pltpu.sync_copy(zeros_hbm, acc.at[pl.ds(base, ROWS_PER_TILE)])
    plsc.subcore_barrier()

    @pl.loop(0, CHUNKS_PER_TILE, step=NI)
    def _edges(g):
        for i in range(NI):
            chunk = g + i
            b = i % NR

            @pl.when(chunk >= NR)
            def _():
                pltpu.make_async_copy(
                    bufs[b], acc.at[_scatter_idx(idx_v, (i + NR) % NI)],
                    semw[b]).wait()

            for d in _idx_copies(eidx_hbm, wid, chunk, idx_v, i,
                                 semi[i], semi2[i]):
                d.wait()

            @pl.when(_gid(wid, chunk) >= TOTAL_CHUNKS)
            def _():
                _mask_invalid(idx_v, i)

            pltpu.async_copy(msg_hbm.at[_gather_idx(idx_v, i)], bufs[b],
                             semr[b])

            @pl.when(chunk >= LAG)
            def _():
                bs = (i + 1) % NR
                qs = (i + NI - LAG) % NI
                pltpu.make_async_copy(msg_hbm.at[_gather_idx(idx_v, qs)],
                                      bufs[bs], semr[bs]).wait()
                pltpu.async_copy(bufs[bs], acc.at[_scatter_idx(idx_v, qs)],
                                 semw[bs], add=True)

            @pl.when(chunk + NR < CHUNKS_PER_TILE)
            def _():
                q2 = (i + NR) % NI
                for d in _idx_copies(eidx_hbm, wid, chunk + NR, idx_v, q2,
                                     semi[q2], semi2[q2]):
                    d.start()

    for t in range(LAG):
        ct = CHUNKS_PER_TILE - LAG + t
        bs = ct % NR
        qs = ct % NI
        pltpu.make_async_copy(msg_hbm.at[_gather_idx(idx_v, qs)],
                              bufs[bs], semr[bs]).wait()
        pltpu.sync_copy(bufs[bs], acc.at[_scatter_idx(idx_v, qs)], add=True)
    cl = CHUNKS_PER_TILE - LAG - 1
    pltpu.make_async_copy(bufs[cl % NR],
                          acc.at[_scatter_idx(idx_v, cl % NI)],
                          semw[cl % NR]).wait()
    plsc.subcore_barrier()

    @pl.when(base + ROWS_PER_TILE <= N)
    def _():
        pltpu.sync_copy(acc.at[pl.ds(base, ROWS_PER_TILE)],
                        outp_hbm.at[c, pl.ds(base, ROWS_PER_TILE)])

    @pl.when(base + ROWS_PER_TILE > N)
    def _():
        pltpu.sync_copy(acc.at[pl.ds(base, LAST_ROWS)],
                        outp_hbm.at[c, pl.ds(base, LAST_ROWS)])


def _sc_aggregate(node_msg, eidx, zeros):
    mesh = plsc.VectorSubcoreMesh(core_axis_name="c", subcore_axis_name="s",
                                  num_cores=NC)
    k = pl.kernel(
        _sc_body,
        mesh=mesh,
        out_type=jax.ShapeDtypeStruct((NC, N, D), jnp.float32),
        scratch_types=[
            pltpu.VMEM((NI, 2, CHUNK), jnp.int32),
            [pltpu.VMEM((CHUNK, D), jnp.float32)] * NR,
            pltpu.VMEM_SHARED((ACC_ROWS, D), jnp.float32),
            [pltpu.SemaphoreType.DMA] * NI,
            [pltpu.SemaphoreType.DMA] * NI,
            [pltpu.SemaphoreType.DMA] * NR,
            [pltpu.SemaphoreType.DMA] * NR,
        ],
    )
    return k(node_msg, eidx, zeros)


def _add_body(p_ref, o_ref):
    o_ref[...] = p_ref[0] + p_ref[1]


def _reduce_partials(partials):
    blk = 1000
    return pl.pallas_call(
        _add_body,
        grid=(N // blk,),
        in_specs=[pl.BlockSpec((NC, blk, D), lambda i: (0, i, 0))],
        out_specs=pl.BlockSpec((blk, D), lambda i: (i, 0)),
        out_shape=jax.ShapeDtypeStruct((N, D), jnp.float32),
    )(partials)


def kernel(x, scalars, edge_index, W1, b1, W2, b2):
    node_msg, zeros = _node_mlp(x, scalars, W1, b1, W2, b2)
    eidx = edge_index.astype(jnp.int32)
    return _reduce_partials(_sc_aggregate(node_msg, eidx, zeros))
